# initial kernel scaffold (unmeasured)
import jax
import jax.numpy as jnp
from jax import lax
from jax.experimental import pallas as pl
from jax.experimental.pallas import tpu as pltpu

N_DEV = 16
T = 512
D = 256
H = 512
E = 64
E_LOC = E // N_DEV
CAP = 6
T_LOC = T // N_DEV


def kernel(x, router_W, route_idx, expert_W):
    def body(x_ref, rw_ref, ridx_ref, w_ref, out_ref,
             partial_ref, recv_ref, send_sems, recv_sems):
        my = lax.axis_index("i")

        ridx = ridx_ref[...]
        oh = (ridx == lax.broadcasted_iota(jnp.int32, (T, E), 1)).astype(
            jnp.float32)
        same = lax.dot_general(oh, oh, (((1,), (1,)), ((), ())),
                               preferred_element_type=jnp.float32)
        row = lax.broadcasted_iota(jnp.int32, (T, T), 0)
        col = lax.broadcasted_iota(jnp.int32, (T, T), 1)
        tri = (col < row).astype(jnp.float32)
        rank = jnp.sum(same * tri, axis=1, keepdims=True)
        keep = rank < float(CAP)

        xb = x_ref[...].astype(jnp.bfloat16)
        acc = jnp.zeros((T, H), jnp.float32)
        for el in range(E_LOC):
            e = my * E_LOC + el
            m = keep & (ridx == e)
            xm = jnp.where(m, xb, jnp.bfloat16(0.0))
            acc = acc + jnp.dot(xm, w_ref[el].astype(jnp.bfloat16),
                                preferred_element_type=jnp.float32)
        partial_ref[...] = acc.astype(jnp.bfloat16)

        rdmas = []
        for s in range(1, N_DEV):
            j = lax.rem(my + s, N_DEV)
            slot = N_DEV - 1 - s
            rdma = pltpu.make_async_remote_copy(
                src_ref=partial_ref.at[pl.ds(j * T_LOC, T_LOC), :],
                dst_ref=recv_ref.at[slot],
                send_sem=send_sems.at[slot],
                recv_sem=recv_sems.at[slot],
                device_id=(j,),
                device_id_type=pl.DeviceIdType.MESH,
            )
            rdma.start()
            rdmas.append(rdma)

        acc_out = partial_ref[pl.ds(my * T_LOC, T_LOC), :].astype(jnp.float32)
        for rdma in rdmas:
            rdma.wait_recv()
        for slot in range(N_DEV - 1):
            acc_out = acc_out + recv_ref[slot].astype(jnp.float32)
        out_ref[...] = acc_out
        for rdma in rdmas:
            rdma.wait_send()

    return pl.pallas_call(
        body,
        out_shape=jax.ShapeDtypeStruct((T_LOC, H), jnp.float32),
        in_specs=[pl.BlockSpec(memory_space=pltpu.VMEM)] * 4,
        out_specs=pl.BlockSpec(memory_space=pltpu.VMEM),
        scratch_shapes=[
            pltpu.VMEM((T, H), jnp.bfloat16),
            pltpu.VMEM((N_DEV - 1, T_LOC, H), jnp.bfloat16),
            pltpu.SemaphoreType.DMA((N_DEV - 1,)),
            pltpu.SemaphoreType.DMA((N_DEV - 1,)),
        ],
        compiler_params=pltpu.CompilerParams(collective_id=0),
    )(x, router_W, route_idx, expert_W)


# baseline (device time: 21897 ns/iter reference)
import jax
import jax.numpy as jnp
from jax import lax
from jax.experimental import pallas as pl
from jax.experimental.pallas import tpu as pltpu

N_DEV = 16
T = 512
D = 256
H = 512
E = 64
E_LOC = E // N_DEV
CAP = 6
T_LOC = T // N_DEV


def kernel(x, router_W, route_idx, expert_W):
    def body(x_ref, rw_ref, ridx_ref, w_ref, out_ref,
             partial_ref, recv_ref, send_sems, recv_sems):
        my = lax.axis_index("i")

        ridx = ridx_ref[...]
        oh = (ridx == lax.broadcasted_iota(jnp.int32, (T, E), 1)).astype(
            jnp.float32)
        same = lax.dot_general(oh, oh, (((1,), (1,)), ((), ())),
                               preferred_element_type=jnp.float32)
        row = lax.broadcasted_iota(jnp.int32, (T, T), 0)
        col = lax.broadcasted_iota(jnp.int32, (T, T), 1)
        tri = (col < row).astype(jnp.float32)
        rank = jnp.sum(same * tri, axis=1, keepdims=True)
        keep = rank < float(CAP)

        xb = x_ref[...].astype(jnp.bfloat16)
        acc = jnp.zeros((T, H), jnp.float32)
        for el in range(E_LOC):
            e = my * E_LOC + el
            m = keep & (ridx == e)
            xm = jnp.where(m, xb, jnp.bfloat16(0.0))
            acc = acc + jnp.dot(xm, w_ref[el].astype(jnp.bfloat16),
                                preferred_element_type=jnp.float32)
        partial_ref[...] = acc.astype(jnp.bfloat16)

        rdmas = []
        for s in range(1, N_DEV):
            j = lax.rem(my + s, N_DEV)
            slot = N_DEV - 1 - s
            rdma = pltpu.make_async_remote_copy(
                src_ref=partial_ref.at[pl.ds(j * T_LOC, T_LOC), :],
                dst_ref=recv_ref.at[slot],
                send_sem=send_sems.at[slot],
                recv_sem=recv_sems.at[slot],
                device_id=(j,),
                device_id_type=pl.DeviceIdType.MESH,
            )
            rdma.start()
            rdmas.append(rdma)

        acc_out = partial_ref[pl.ds(my * T_LOC, T_LOC), :].astype(jnp.float32)
        for rdma in rdmas:
            rdma.wait_recv()
        for slot in range(N_DEV - 1):
            acc_out = acc_out + recv_ref[slot].astype(jnp.float32)
        out_ref[...] = acc_out
        for rdma in rdmas:
            rdma.wait_send()

    return pl.pallas_call(
        body,
        out_shape=jax.ShapeDtypeStruct((T_LOC, H), jnp.float32),
        in_specs=[pl.BlockSpec(memory_space=pltpu.VMEM)] * 4,
        out_specs=pl.BlockSpec(memory_space=pltpu.VMEM),
        scratch_shapes=[
            pltpu.VMEM((T, H), jnp.bfloat16),
            pltpu.VMEM((N_DEV - 1, T_LOC, H), jnp.bfloat16),
            pltpu.SemaphoreType.DMA((N_DEV - 1,)),
            pltpu.SemaphoreType.DMA((N_DEV - 1,)),
        ],
    )(x, router_W, route_idx, expert_W)


# device time: 15695 ns/iter; 1.3952x vs baseline; 1.3952x over previous
import jax
import jax.numpy as jnp
from jax import lax
from jax.experimental import pallas as pl
from jax.experimental.pallas import tpu as pltpu

N_DEV = 16
T = 512
D = 256
H = 512
E = 64
E_LOC = E // N_DEV
CAP = 6
CAP_PAD = 8
T_LOC = T // N_DEV
N_PACK = E_LOC * CAP
N_STAGE = E_LOC * CAP_PAD
G = E * CAP


def kernel(x, router_W, route_idx, expert_W):
    def body(x_hbm, rw_hbm, ridx_ref, w_hbm, out_ref,
             x_ref, w_ref, stage_ref, pack_ref, recv_ref, rank_ref,
             send_sems, recv_sems, loc_sem, in_sems):
        my = lax.axis_index("i")

        barrier_sem = pltpu.get_barrier_semaphore()
        for s in range(1, N_DEV):
            pl.semaphore_signal(
                barrier_sem, inc=1,
                device_id=(lax.rem(my + s, N_DEV),),
                device_id_type=pl.DeviceIdType.MESH,
            )

        cp_x = pltpu.make_async_copy(x_hbm, x_ref, in_sems.at[0])
        cp_w = pltpu.make_async_copy(w_hbm, w_ref, in_sems.at[1])
        cp_x.start()
        cp_w.start()

        ridx = ridx_ref[...]
        oh = (ridx == lax.broadcasted_iota(jnp.int32, (T, E), 1)).astype(
            jnp.float32)
        same = lax.dot_general(oh, oh, (((1,), (1,)), ((), ())),
                               preferred_element_type=jnp.float32)
        row = lax.broadcasted_iota(jnp.int32, (T, T), 0)
        col = lax.broadcasted_iota(jnp.int32, (T, T), 1)
        tri = (col < row).astype(jnp.float32)
        rank = jnp.sum(same * tri, axis=1, keepdims=True).astype(
            jnp.int32)
        rank_ref[...] = rank

        col32 = lax.broadcasted_iota(jnp.int32, (1, N_STAGE), 1)
        e_cols = my * E_LOC + col32 // CAP_PAD
        c_cols = col32 % CAP_PAD
        mask32 = ((ridx == e_cols) & (rank == c_cols) &
                  (c_cols < CAP)).astype(jnp.float32)
        cp_x.wait()
        xg = lax.dot_general(mask32, x_ref[...], (((0,), (0,)), ((), ())),
                             preferred_element_type=jnp.float32)
        xgb = xg.astype(jnp.bfloat16)
        cp_w.wait()
        for el in range(E_LOC):
            r0 = el * CAP_PAD
            blk = jnp.dot(xgb[r0:r0 + CAP_PAD, :],
                          w_ref[el].astype(jnp.bfloat16),
                          preferred_element_type=jnp.float32)
            stage_ref[r0:r0 + CAP_PAD, :] = blk.astype(jnp.bfloat16)

        prow = lax.broadcasted_iota(jnp.int32, (N_PACK, N_STAGE), 0)
        pcol = lax.broadcasted_iota(jnp.int32, (N_PACK, N_STAGE), 1)
        packm = (pcol == (prow // CAP) * CAP_PAD + prow % CAP).astype(
            jnp.bfloat16)
        pack_ref[...] = lax.dot_general(
            packm, stage_ref[...], (((1,), (0,)), ((), ())),
            preferred_element_type=jnp.float32).astype(jnp.bfloat16)

        pl.semaphore_wait(barrier_sem, N_DEV - 1)

        my_rows = pl.ds(pl.multiple_of(my * N_PACK, 8), N_PACK)
        rdmas = []
        for s in range(1, N_DEV):
            j = lax.rem(my + s, N_DEV)
            rdma = pltpu.make_async_remote_copy(
                src_ref=pack_ref.at[:, :],
                dst_ref=recv_ref.at[my_rows, :],
                send_sem=send_sems.at[s - 1],
                recv_sem=recv_sems.at[s - 1],
                device_id=(j,),
                device_id_type=pl.DeviceIdType.MESH,
            )
            rdma.start()
            rdmas.append(rdma)
        loc = pltpu.make_async_copy(pack_ref, recv_ref.at[my_rows, :],
                                    loc_sem)
        loc.start()

        my0 = pl.multiple_of(my * T_LOC, T_LOC)
        rank_mine = rank_ref[pl.ds(my0, T_LOC), :]
        ridx_mine = ridx_ref[pl.ds(my0, T_LOC), :]
        colg = lax.broadcasted_iota(jnp.int32, (T_LOC, G), 1)
        sel = ((ridx_mine == colg // CAP) & (rank_mine == colg % CAP)
               ).astype(jnp.bfloat16)

        loc.wait()
        for s in range(1, N_DEV):
            src_dev = lax.rem(my + (N_DEV - s), N_DEV)
            recv = pltpu.make_async_remote_copy(
                src_ref=pack_ref.at[:, :],
                dst_ref=recv_ref.at[
                    pl.ds(pl.multiple_of(src_dev * N_PACK, 8), N_PACK), :],
                send_sem=send_sems.at[s - 1],
                recv_sem=recv_sems.at[s - 1],
                device_id=(my,),
                device_id_type=pl.DeviceIdType.MESH,
            )
            recv.wait_recv()

        out_ref[...] = lax.dot_general(
            sel, recv_ref[...], (((1,), (0,)), ((), ())),
            preferred_element_type=jnp.float32)

        for rdma in rdmas:
            rdma.wait_send()

    return pl.pallas_call(
        body,
        out_shape=jax.ShapeDtypeStruct((T_LOC, H), jnp.float32),
        in_specs=[
            pl.BlockSpec(memory_space=pl.ANY),
            pl.BlockSpec(memory_space=pl.ANY),
            pl.BlockSpec(memory_space=pltpu.VMEM),
            pl.BlockSpec(memory_space=pl.ANY),
        ],
        out_specs=pl.BlockSpec(memory_space=pltpu.VMEM),
        scratch_shapes=[
            pltpu.VMEM((T, D), jnp.float32),
            pltpu.VMEM((E_LOC, D, H), jnp.float32),
            pltpu.VMEM((N_STAGE, H), jnp.bfloat16),
            pltpu.VMEM((N_PACK, H), jnp.bfloat16),
            pltpu.VMEM((G, H), jnp.bfloat16),
            pltpu.VMEM((T, 1), jnp.int32),
            pltpu.SemaphoreType.DMA((N_DEV - 1,)),
            pltpu.SemaphoreType.DMA((N_DEV - 1,)),
            pltpu.SemaphoreType.DMA,
            pltpu.SemaphoreType.DMA((2,)),
        ],
        compiler_params=pltpu.CompilerParams(collective_id=0),
    )(x, router_W, route_idx, expert_W)


# device time: 15634 ns/iter; 1.4006x vs baseline; 1.0039x over previous
import jax
import jax.numpy as jnp
from jax import lax
from jax.experimental import pallas as pl
from jax.experimental.pallas import tpu as pltpu

N_DEV = 16
T = 512
D = 256
H = 512
E = 64
E_LOC = E // N_DEV
CAP = 6
CAP_PAD = 8
T_LOC = T // N_DEV
N_PACK = E_LOC * CAP
N_STAGE = E_LOC * CAP_PAD
G = E * CAP


def kernel(x, router_W, route_idx, expert_W):
    def body(x_ref, rw_ref, ridx_ref, w_ref, out_ref,
             stage_ref, pack_ref, recv_ref, rank_ref,
             send_sems, recv_sems, loc_sem):
        my = lax.axis_index("i")

        barrier_sem = pltpu.get_barrier_semaphore()
        for s in range(1, N_DEV):
            pl.semaphore_signal(
                barrier_sem, inc=1,
                device_id=(lax.rem(my + s, N_DEV),),
                device_id_type=pl.DeviceIdType.MESH,
            )

        ridx = ridx_ref[...]
        oh = (ridx == lax.broadcasted_iota(jnp.int32, (T, E), 1)).astype(
            jnp.bfloat16)
        same = lax.dot_general(oh, oh, (((1,), (1,)), ((), ())),
                               preferred_element_type=jnp.float32)
        row = lax.broadcasted_iota(jnp.int32, (T, T), 0)
        col = lax.broadcasted_iota(jnp.int32, (T, T), 1)
        tri = (col < row).astype(jnp.float32)
        rank = jnp.sum(same * tri, axis=1, keepdims=True).astype(
            jnp.int32)
        rank_ref[...] = rank

        col32 = lax.broadcasted_iota(jnp.int32, (1, N_STAGE), 1)
        e_cols = my * E_LOC + col32 // CAP_PAD
        c_cols = col32 % CAP_PAD
        mask32 = ((ridx == e_cols) & (rank == c_cols) &
                  (c_cols < CAP)).astype(jnp.bfloat16)
        xgb = lax.dot_general(mask32, x_ref[...].astype(jnp.bfloat16),
                              (((0,), (0,)), ((), ())),
                              preferred_element_type=jnp.float32
                              ).astype(jnp.bfloat16)
        for el in range(E_LOC):
            r0 = el * CAP_PAD
            blk = jnp.dot(xgb[r0:r0 + CAP_PAD, :],
                          w_ref[el].astype(jnp.bfloat16),
                          preferred_element_type=jnp.float32)
            stage_ref[r0:r0 + CAP_PAD, :] = blk.astype(jnp.bfloat16)

        prow = lax.broadcasted_iota(jnp.int32, (N_PACK, N_STAGE), 0)
        pcol = lax.broadcasted_iota(jnp.int32, (N_PACK, N_STAGE), 1)
        packm = (pcol == (prow // CAP) * CAP_PAD + prow % CAP).astype(
            jnp.bfloat16)
        pack_ref[...] = lax.dot_general(
            packm, stage_ref[...], (((1,), (0,)), ((), ())),
            preferred_element_type=jnp.float32).astype(jnp.bfloat16)

        my_rows = pl.ds(pl.multiple_of(my * N_PACK, 8), N_PACK)
        loc = pltpu.make_async_copy(pack_ref, recv_ref.at[my_rows, :],
                                    loc_sem)
        loc.start()

        pl.semaphore_wait(barrier_sem, N_DEV - 1)

        rdmas = []
        for s in range(1, N_DEV):
            j = lax.rem(my + s, N_DEV)
            rdma = pltpu.make_async_remote_copy(
                src_ref=pack_ref.at[:, :],
                dst_ref=recv_ref.at[my_rows, :],
                send_sem=send_sems.at[s - 1],
                recv_sem=recv_sems.at[s - 1],
                device_id=(j,),
                device_id_type=pl.DeviceIdType.MESH,
            )
            rdma.start()
            rdmas.append(rdma)

        my0 = pl.multiple_of(my * T_LOC, T_LOC)
        rank_mine = rank_ref[pl.ds(my0, T_LOC), :]
        ridx_mine = ridx_ref[pl.ds(my0, T_LOC), :]
        colg = lax.broadcasted_iota(jnp.int32, (T_LOC, G), 1)
        sel = ((ridx_mine == colg // CAP) & (rank_mine == colg % CAP)
               ).astype(jnp.bfloat16)

        loc.wait()
        for s in range(1, N_DEV):
            src_dev = lax.rem(my + (N_DEV - s), N_DEV)
            recv = pltpu.make_async_remote_copy(
                src_ref=pack_ref.at[:, :],
                dst_ref=recv_ref.at[
                    pl.ds(pl.multiple_of(src_dev * N_PACK, 8), N_PACK), :],
                send_sem=send_sems.at[s - 1],
                recv_sem=recv_sems.at[s - 1],
                device_id=(my,),
                device_id_type=pl.DeviceIdType.MESH,
            )
            recv.wait_recv()

        out_ref[...] = lax.dot_general(
            sel, recv_ref[...], (((1,), (0,)), ((), ())),
            preferred_element_type=jnp.float32)

        for rdma in rdmas:
            rdma.wait_send()

    return pl.pallas_call(
        body,
        out_shape=jax.ShapeDtypeStruct((T_LOC, H), jnp.float32),
        in_specs=[pl.BlockSpec(memory_space=pltpu.VMEM)] * 4,
        out_specs=pl.BlockSpec(memory_space=pltpu.VMEM),
        scratch_shapes=[
            pltpu.VMEM((N_STAGE, H), jnp.bfloat16),
            pltpu.VMEM((N_PACK, H), jnp.bfloat16),
            pltpu.VMEM((G, H), jnp.bfloat16),
            pltpu.VMEM((T, 1), jnp.int32),
            pltpu.SemaphoreType.DMA((N_DEV - 1,)),
            pltpu.SemaphoreType.DMA((N_DEV - 1,)),
            pltpu.SemaphoreType.DMA,
        ],
        compiler_params=pltpu.CompilerParams(collective_id=0),
    )(x, router_W, route_idx, expert_W)
